# baseline (speedup 1.0000x reference)
import jax
import jax.numpy as jnp
from jax.experimental import pallas as pl
from jax.experimental.pallas import tpu as pltpu

_B = 4
_N = 256
_E1 = (_N * (_N - 1)) // 2
_NCH = 5
_C = _E1 // _NCH


def _edge_index_body(out_ref, base_ref):
    k = pl.program_id(0)

    @pl.when(k == 0)
    def _first():
        sf = jax.lax.broadcasted_iota(jnp.int32, (2, _C), 0).astype(jnp.float32)
        for c in range(_NCH):
            e0 = c * _C
            el = jax.lax.broadcasted_iota(jnp.int32, (2, _C), 1).astype(jnp.float32)
            x = (8.0 * (_E1 - e0) - 6.5) - 8.0 * el
            s = x * jax.lax.rsqrt(x)
            t = jnp.floor(0.5 * s - 0.5)
            rowf = (_N - 2.0) - t
            d = t * (0.5 * t + 1.5) + (el + (2.0 - _E1 + e0))
            v = (rowf + sf * d).astype(jnp.int32)
            base_ref[:, e0:e0 + _C] = v
            out_ref[:, e0:e0 + _C] = v

    @pl.when(k != 0)
    def _rest():
        out_ref[:, :] = base_ref[:, :] + k * _N


def kernel(x_topology, x_temporal, W_gnn, b_gnn, W_mean, b_mean, W_var, b_var, W_w, b_w):
    return pl.pallas_call(
        _edge_index_body,
        grid=(_B,),
        out_specs=pl.BlockSpec((2, _E1), lambda k: (0, k)),
        out_shape=jax.ShapeDtypeStruct((2, _B * _E1), jnp.int32),
        scratch_shapes=[pltpu.VMEM((2, _E1), jnp.int32)],
        compiler_params=pltpu.CompilerParams(dimension_semantics=("arbitrary",)),
    )()


# speedup vs baseline: 1.2745x; 1.2745x over previous
import jax
import jax.numpy as jnp
from jax.experimental import pallas as pl
from jax.experimental.pallas import tpu as pltpu

_B = 4
_N = 256
_E1 = (_N * (_N - 1)) // 2
_NCH = 5
_C = _E1 // _NCH


def _edge_index_body(out_ref, base_ref):
    j = pl.program_id(0)

    @pl.when(j == 0)
    def _first():
        sf = jax.lax.broadcasted_iota(jnp.int32, (2, _C), 0).astype(jnp.float32)
        for c in range(_NCH):
            e0 = c * _C
            el = jax.lax.broadcasted_iota(jnp.int32, (2, _C), 1).astype(jnp.float32)
            x = (8.0 * (_E1 - e0) - 6.5) - 8.0 * el
            s = x * jax.lax.rsqrt(x)
            t = jnp.floor(0.5 * s - 0.5)
            rowf = (_N - 2.0) - t
            d = t * (0.5 * t + 1.5) + (el + (2.0 - _E1 + e0))
            v = (rowf + sf * d).astype(jnp.int32)
            base_ref[:, e0:e0 + _C] = v
            out_ref[:, e0:e0 + _C] = v
            out_ref[:, _E1 + e0:_E1 + e0 + _C] = v + _N

    @pl.when(j == 1)
    def _second():
        for c in range(_NCH):
            e0 = c * _C
            v = base_ref[:, e0:e0 + _C]
            out_ref[:, e0:e0 + _C] = v + 2 * _N
            out_ref[:, _E1 + e0:_E1 + e0 + _C] = v + 3 * _N


def kernel(x_topology, x_temporal, W_gnn, b_gnn, W_mean, b_mean, W_var, b_var, W_w, b_w):
    return pl.pallas_call(
        _edge_index_body,
        grid=(2,),
        out_specs=pl.BlockSpec((2, 2 * _E1), lambda j: (0, j)),
        out_shape=jax.ShapeDtypeStruct((2, _B * _E1), jnp.int32),
        scratch_shapes=[pltpu.VMEM((2, _E1), jnp.int32)],
        compiler_params=pltpu.CompilerParams(dimension_semantics=("arbitrary",)),
    )()


# speedup vs baseline: 1.3065x; 1.0251x over previous
import jax
import jax.numpy as jnp
from jax.experimental import pallas as pl
from jax.experimental.pallas import tpu as pltpu

_B = 4
_N = 256
_E1 = (_N * (_N - 1)) // 2
_NCH = 5
_C = _E1 // _NCH


def _edge_index_body(out_ref, vbuf_ref, sem):
    sf = jax.lax.broadcasted_iota(jnp.int32, (2, _C), 0).astype(jnp.float32)
    copies = []
    for c in range(_NCH):
        e0 = c * _C
        el = jax.lax.broadcasted_iota(jnp.int32, (2, _C), 1).astype(jnp.float32)
        x = (8.0 * (_E1 - e0) - 6.5) - 8.0 * el
        s = x * jax.lax.rsqrt(x)
        t = jnp.floor(0.5 * s - 0.5)
        rowf = (_N - 2.0) - t
        d = t * (0.5 * t + 1.5) + (el + (2.0 - _E1 + e0))
        v = (rowf + sf * d).astype(jnp.int32)
        for k in range(_B):
            lo = k * _E1 + e0
            vbuf_ref[:, lo:lo + _C] = v if k == 0 else v + (k * _N)
            cp = pltpu.make_async_copy(
                vbuf_ref.at[:, lo:lo + _C], out_ref.at[:, lo:lo + _C], sem)
            cp.start()
            copies.append(cp)
    for cp in copies:
        cp.wait()


def kernel(x_topology, x_temporal, W_gnn, b_gnn, W_mean, b_mean, W_var, b_var, W_w, b_w):
    return pl.pallas_call(
        _edge_index_body,
        out_specs=pl.BlockSpec(memory_space=pl.ANY),
        out_shape=jax.ShapeDtypeStruct((2, _B * _E1), jnp.int32),
        scratch_shapes=[pltpu.VMEM((2, _B * _E1), jnp.int32),
                        pltpu.SemaphoreType.DMA],
    )()
